# baseline (speedup 1.0000x reference)
import functools

import jax
import jax.numpy as jnp
from jax import lax
from jax.experimental import pallas as pl
from jax.experimental.pallas import tpu as pltpu
from jax.experimental.pallas import tpu_sc as plsc

NC = 2
NS = 16
NW = NC * NS

LANES = 16
GRP = 16
PSTEP = 512
GPS = PSTEP // GRP


def _pair_table_body(e_ref, w_ref, b_ref, o_ref):
    h = (
        jnp.dot(e_ref[...], w_ref[...], preferred_element_type=jnp.float32)
        + b_ref[...]
    )
    v = h.shape[0]
    d = h.shape[1]
    a = jnp.broadcast_to(h[:, None, :], (v, v, d))
    bb = jnp.broadcast_to(h[None, :, :], (v, v, d))
    o_ref[...] = jnp.concatenate([a, bb], axis=-1)


def _make_sc_gather(n, v, d):
    d2 = 2 * d
    assert d2 == LANES
    npair = n // 2
    per_w = npair // NW
    nstep = per_w // PSTEP
    ntok_w = per_w * 2
    orows_step = PSTEP * d2 // 128
    orow_w = per_w * d2 // 128
    out_rows = npair * d2 // 128
    tsz = v * v * d2
    assert npair * 2 == n and per_w * NW == npair and nstep * PSTEP == per_w

    mesh = plsc.VectorSubcoreMesh(core_axis_name="c", subcore_axis_name="s")

    @functools.partial(
        pl.kernel,
        out_type=jax.ShapeDtypeStruct((out_rows, 128), jnp.float32),
        mesh=mesh,
        scratch_types=[
            pltpu.VMEM((ntok_w,), jnp.int32),
            pltpu.VMEM((tsz,), jnp.float32),
            pltpu.VMEM((orows_step, 128), jnp.float32),
        ],
        compiler_params=pltpu.CompilerParams(
            use_tc_tiling_on_sc=False, needs_layout_passes=False
        ),
    )
    def sc_gather(x_hbm, t2_hbm, out_hbm, xbig, t2t, rows):
        wid = lax.axis_index("s") * NC + lax.axis_index("c")
        pltpu.sync_copy(t2_hbm, t2t)
        pltpu.sync_copy(x_hbm.at[pl.ds(wid * ntok_w, ntok_w)], xbig)

        iota = lax.iota(jnp.int32, LANES)
        iota2 = iota * 2
        r1base = (iota & 7) * d2
        rhalf = iota >> 3
        obase = wid * orow_w

        def step(s, carry):
            @plsc.parallel_loop(0, GPS, unroll=2)
            def group(g):
                tbase = (s * PSTEP + g * GRP) * 2
                te = tbase + iota2
                ev = plsc.load_gather(xbig, [te])
                od = plsc.load_gather(xbig, [te + 1])
                p16 = (ev * v + od) * d2
                r0v = rhalf + g * 2
                for c in range(d2):
                    vals = plsc.load_gather(t2t, [p16 + c])
                    plsc.store_scatter(rows, [r0v, r1base + c], vals)

            pltpu.sync_copy(
                rows, out_hbm.at[pl.ds(obase + s * orows_step, orows_step)]
            )
            return carry

        lax.fori_loop(0, nstep, step, 0)

    return sc_gather


def kernel(x, embed_table, W, b):
    bs, sl = x.shape
    n = bs * sl
    v = embed_table.shape[0]
    d = W.shape[1]
    t2 = pl.pallas_call(
        _pair_table_body,
        out_shape=jax.ShapeDtypeStruct((v, v, 2 * d), jnp.float32),
    )(embed_table, W, b.reshape(1, d))
    xf = lax.optimization_barrier(x.reshape(n))
    out = _make_sc_gather(n, v, d)(xf, t2.reshape(v * v * 2 * d))
    return lax.optimization_barrier(out).reshape(bs, sl, d)
